# Initial kernel scaffold; baseline (speedup 1.0000x reference)
import jax
import jax.numpy as jnp
from jax import lax
from jax.experimental import pallas as pl
from jax.experimental.pallas import tpu as pltpu


def _body(q_ref, w_ref, out_ref, loss_ref, p_ref):
    b = pl.program_id(0)

    @pl.when(b == 0)
    def _build_perm():
        mi = lax.broadcasted_iota(jnp.int32, (1024, 1024), 0)
        pi = lax.broadcasted_iota(jnp.int32, (1024, 1024), 1)
        pt = (pi % 32) * 32 + pi // 32
        p_ref[...] = (mi == pt).astype(jnp.float32)

    q = q_ref[0]
    w = w_ref[...]

    ab = lax.dot_general(w, q, (((1,), (0,)), ((), ())),
                         preferred_element_type=jnp.float32)
    b2 = jnp.sum(w * w, axis=1, keepdims=True)
    a2 = jnp.sum(q * q, axis=0, keepdims=True)
    d2 = jnp.maximum(a2 + b2 - 2.0 * ab, 0.0)

    m = jnp.min(d2, axis=0, keepdims=True)
    kio = lax.broadcasted_iota(jnp.int32, (128, 1024), 0)
    idx = jnp.min(jnp.where(d2 == m, kio, jnp.int32(128)),
                  axis=0, keepdims=True)
    onehot = (kio == idx).astype(jnp.float32)

    onehot_t = lax.dot_general(onehot, p_ref[...], (((1,), (0,)), ((), ())),
                               preferred_element_type=jnp.float32)
    outb = lax.dot_general(w, onehot_t, (((0,), (0,)), ((), ())),
                           preferred_element_type=jnp.float32,
                           precision=lax.Precision.HIGHEST)
    out_ref[0] = outb
    loss_ref[0, 0] = jnp.sum(m)


def kernel(pre_quantized, W):
    q = pre_quantized.reshape(16, 256, 1024)
    out_flat, loss_parts = pl.pallas_call(
        _body,
        grid=(16,),
        in_specs=[
            pl.BlockSpec((1, 256, 1024), lambda b: (b, 0, 0)),
            pl.BlockSpec((128, 256), lambda b: (0, 0)),
        ],
        out_specs=[
            pl.BlockSpec((1, 256, 1024), lambda b: (b, 0, 0)),
            pl.BlockSpec((1, 1), lambda b: (b, 0), memory_space=pltpu.SMEM),
        ],
        out_shape=[
            jax.ShapeDtypeStruct((16, 256, 1024), jnp.float32),
            jax.ShapeDtypeStruct((16, 1), jnp.float32),
        ],
        scratch_shapes=[pltpu.VMEM((1024, 1024), jnp.float32)],
    )(q, W)
    out = out_flat.reshape(16, 256, 32, 32)
    loss = jnp.sum(loss_parts) * (1.25 / 4194304.0)
    return out, loss


# speedup vs baseline: 1.9301x; 1.9301x over previous
import jax
import jax.numpy as jnp
from jax import lax
from jax.experimental import pallas as pl
from jax.experimental.pallas import tpu as pltpu


def _body(q_ref, w_ref, out_ref, loss_ref, p_ref):
    b = pl.program_id(0)

    @pl.when(b == 0)
    def _build_perm():
        mi = lax.broadcasted_iota(jnp.int32, (1024, 1024), 0)
        pi = lax.broadcasted_iota(jnp.int32, (1024, 1024), 1)
        pt = (pi % 32) * 32 + pi // 32
        p_ref[...] = (mi == pt).astype(jnp.float32)

    q = q_ref[0]
    w = w_ref[...]

    ab = lax.dot_general(w, q, (((1,), (0,)), ((), ())),
                         preferred_element_type=jnp.float32)
    b2 = jnp.sum(w * w, axis=1, keepdims=True)
    a2 = jnp.sum(q * q, axis=0, keepdims=True)
    d2 = jnp.maximum(a2 + b2 - 2.0 * ab, 0.0)

    m = jnp.min(d2, axis=0, keepdims=True)
    kio = lax.broadcasted_iota(jnp.int32, (128, 1024), 0)
    idx = jnp.min(jnp.where(d2 == m, kio, jnp.int32(128)),
                  axis=0, keepdims=True)
    onehot = (kio == idx).astype(jnp.float32)

    onehot_t = lax.dot_general(onehot, p_ref[...], (((1,), (0,)), ((), ())),
                               preferred_element_type=jnp.float32)
    outb = lax.dot_general(w, onehot_t, (((0,), (0,)), ((), ())),
                           preferred_element_type=jnp.float32,
                           precision=lax.Precision.HIGHEST)
    out_ref[0] = outb
    loss_ref[0, 0, 0] = jnp.sum(m)


def kernel(pre_quantized, W):
    q = pre_quantized.reshape(16, 256, 1024)
    out_flat, loss_parts = pl.pallas_call(
        _body,
        grid=(16,),
        in_specs=[
            pl.BlockSpec((1, 256, 1024), lambda b: (b, 0, 0)),
            pl.BlockSpec((128, 256), lambda b: (0, 0)),
        ],
        out_specs=[
            pl.BlockSpec((1, 256, 1024), lambda b: (b, 0, 0)),
            pl.BlockSpec((1, 1, 1), lambda b: (b, 0, 0), memory_space=pltpu.SMEM),
        ],
        out_shape=[
            jax.ShapeDtypeStruct((16, 256, 1024), jnp.float32),
            jax.ShapeDtypeStruct((16, 1, 1), jnp.float32),
        ],
        scratch_shapes=[pltpu.VMEM((1024, 1024), jnp.float32)],
    )(q, W)
    out = out_flat.reshape(16, 256, 32, 32)
    loss = jnp.sum(loss_parts) * (1.25 / 4194304.0)
    return out, loss


# speedup vs baseline: 2.1291x; 1.1031x over previous
import jax
import jax.numpy as jnp
from jax import lax
from jax.experimental import pallas as pl
from jax.experimental.pallas import tpu as pltpu


def _body(q_ref, w_ref, out_ref, loss_ref, p_ref):
    b = pl.program_id(0)

    @pl.when(b == 0)
    def _build_perm():
        mi = lax.broadcasted_iota(jnp.int32, (1024, 1024), 0)
        pi = lax.broadcasted_iota(jnp.int32, (1024, 1024), 1)
        pt = (pi % 32) * 32 + pi // 32
        p_ref[...] = (mi == pt).astype(jnp.float32)

    q = q_ref[0]
    w = w_ref[...]

    ab = lax.dot_general(w, q, (((1,), (0,)), ((), ())),
                         preferred_element_type=jnp.float32)
    b2 = jnp.sum(w * w, axis=1, keepdims=True)
    a2 = jnp.sum(q * q, axis=0, keepdims=True)
    d2 = jnp.maximum(a2 + b2 - 2.0 * ab, 0.0)

    m = jnp.min(d2, axis=0, keepdims=True)
    kio = lax.broadcasted_iota(jnp.int32, (128, 1024), 0)
    idx = jnp.min(jnp.where(d2 == m, kio, jnp.int32(128)),
                  axis=0, keepdims=True)
    idx_t = lax.dot_general(idx.astype(jnp.float32), p_ref[...],
                            (((1,), (0,)), ((), ())),
                            preferred_element_type=jnp.float32)
    onehot_t = (kio == idx_t.astype(jnp.int32)).astype(jnp.float32)
    outb = lax.dot_general(w, onehot_t, (((0,), (0,)), ((), ())),
                           preferred_element_type=jnp.float32)
    out_ref[0] = outb
    loss_ref[0, 0, 0] = jnp.sum(m)


def kernel(pre_quantized, W):
    q = pre_quantized.reshape(16, 256, 1024)
    out_flat, loss_parts = pl.pallas_call(
        _body,
        grid=(16,),
        in_specs=[
            pl.BlockSpec((1, 256, 1024), lambda b: (b, 0, 0)),
            pl.BlockSpec((128, 256), lambda b: (0, 0)),
        ],
        out_specs=[
            pl.BlockSpec((1, 256, 1024), lambda b: (b, 0, 0)),
            pl.BlockSpec((1, 1, 1), lambda b: (b, 0, 0), memory_space=pltpu.SMEM),
        ],
        out_shape=[
            jax.ShapeDtypeStruct((16, 256, 1024), jnp.float32),
            jax.ShapeDtypeStruct((16, 1, 1), jnp.float32),
        ],
        scratch_shapes=[pltpu.VMEM((1024, 1024), jnp.float32)],
    )(q, W)
    out = out_flat.reshape(16, 256, 32, 32)
    loss = jnp.sum(loss_parts) * (1.25 / 4194304.0)
    return out, loss
